# baseline (speedup 1.0000x reference)
import functools

import jax
import jax.numpy as jnp
from jax import lax
from jax.experimental import pallas as pl
from jax.experimental.pallas import tpu as pltpu
from jax.experimental.pallas import tpu_sc as plsc

NUM_CATS = 1000000
EMBED_DIM = 32
BATCH = 16384
FIELDS = 26

_B = BATCH * FIELDS
_NC = 2
_NS = 16
_NW = _NC * _NS
_BPW = _B // _NW
_CHUNK = 832
_NCHUNK = _BPW // _CHUNK
_NBUF = 4


def _make_gather():
    mesh = plsc.VectorSubcoreMesh(core_axis_name="c", subcore_axis_name="s")

    @functools.partial(
        pl.kernel,
        mesh=mesh,
        compiler_params=pltpu.CompilerParams(use_tc_tiling_on_sc=False),
        out_type=jax.ShapeDtypeStruct((_B, EMBED_DIM), jnp.float32),
        scratch_types=(
            [pltpu.VMEM((_BPW,), jnp.int32)]
            + [pltpu.VMEM((_CHUNK, EMBED_DIM), jnp.float32)] * _NBUF
            + [pltpu.SemaphoreType.DMA] * (2 * _NBUF)
        ),
    )
    def gather_kernel(idx_hbm, table_hbm, out_hbm, idx_v, *rest):
        rows = rest[:_NBUF]
        gsem = rest[_NBUF:2 * _NBUF]
        ssem = rest[2 * _NBUF:]
        wid = lax.axis_index("s") * _NC + lax.axis_index("c")
        base = wid * _BPW
        pltpu.sync_copy(idx_hbm.at[pl.ds(base, _BPW)], idx_v)

        gather = [None] * _NBUF
        store = [None] * _NBUF
        for c in range(min(_NBUF, _NCHUNK)):
            gather[c] = pltpu.async_copy(
                table_hbm.at[idx_v.at[pl.ds(c * _CHUNK, _CHUNK)]],
                rows[c], gsem[c])
        for c in range(_NCHUNK):
            buf = c % _NBUF
            gather[buf].wait()
            store[buf] = pltpu.async_copy(
                rows[buf],
                out_hbm.at[pl.ds(base + c * _CHUNK, _CHUNK)],
                ssem[buf])
            nxt = c + _NBUF
            if nxt < _NCHUNK:
                store[buf].wait()
                gather[buf] = pltpu.async_copy(
                    table_hbm.at[idx_v.at[pl.ds(nxt * _CHUNK, _CHUNK)]],
                    rows[buf], gsem[buf])
        for c in range(max(0, _NCHUNK - _NBUF), _NCHUNK):
            store[c % _NBUF].wait()

    return gather_kernel


_gather = _make_gather()


@jax.jit
def kernel(cat_idx, table):
    flat_idx = cat_idx.reshape(-1).astype(jnp.int32)
    out = _gather(flat_idx, table)
    return out.reshape(BATCH, FIELDS, EMBED_DIM)


# speedup vs baseline: 1.0000x; 1.0000x over previous
import functools

import jax
import jax.numpy as jnp
from jax import lax
from jax.experimental import pallas as pl
from jax.experimental.pallas import tpu as pltpu
from jax.experimental.pallas import tpu_sc as plsc

NUM_CATS = 1000000
EMBED_DIM = 32
BATCH = 16384
FIELDS = 26
_B = BATCH * FIELDS
_NC = 2
_NS = 16
_NW = _NC * _NS
_BPW = _B // _NW
_NSTR = 4
_RNG = _BPW // _NSTR
_CHUNK = 416
_NCH = _RNG // _CHUNK


def _make_gather():
    mesh = plsc.VectorSubcoreMesh(core_axis_name="c", subcore_axis_name="s")

    @functools.partial(
        pl.kernel,
        mesh=mesh,
        compiler_params=pltpu.CompilerParams(use_tc_tiling_on_sc=False),
        out_type=jax.ShapeDtypeStruct((_B, EMBED_DIM), jnp.float32),
        scratch_types=(
            [pltpu.VMEM((_BPW,), jnp.int32)]
            + [pltpu.VMEM((_CHUNK, EMBED_DIM), jnp.float32)] * (2 * _NSTR)
            + [pltpu.SemaphoreType.DMA] * (4 * _NSTR)
        ),
    )
    def gather_kernel(idx_hbm, table_hbm, out_hbm, idx_v, *rest):
        bufs = [rest[2 * s:2 * s + 2] for s in range(_NSTR)]
        sems = rest[2 * _NSTR:]
        gsem = [sems[2 * s:2 * s + 2] for s in range(_NSTR)]
        ssem = [sems[2 * _NSTR + 2 * s:2 * _NSTR + 2 * s + 2]
                for s in range(_NSTR)]
        wid = lax.axis_index("s") * _NC + lax.axis_index("c")
        base = wid * _BPW
        pltpu.sync_copy(idx_hbm.at[pl.ds(base, _BPW)], idx_v)

        gather = [[None, None] for _ in range(_NSTR)]
        store = [[None, None] for _ in range(_NSTR)]

        def off(s, c):
            return s * _RNG + c * _CHUNK

        for b in range(2):
            for s in range(_NSTR):
                gather[s][b] = pltpu.async_copy(
                    table_hbm.at[idx_v.at[pl.ds(off(s, b), _CHUNK)]],
                    bufs[s][b], gsem[s][b])
        for c in range(_NCH):
            b = c % 2
            for s in range(_NSTR):
                gather[s][b].wait()
                store[s][b] = pltpu.async_copy(
                    bufs[s][b],
                    out_hbm.at[pl.ds(base + off(s, c), _CHUNK)],
                    ssem[s][b])
            nxt = c + 2
            if nxt < _NCH:
                for s in range(_NSTR):
                    store[s][b].wait()
                    gather[s][b] = pltpu.async_copy(
                        table_hbm.at[idx_v.at[pl.ds(off(s, nxt), _CHUNK)]],
                        bufs[s][b], gsem[s][b])
        for c in range(_NCH - 2, _NCH):
            for s in range(_NSTR):
                store[s][c % 2].wait()

    return gather_kernel


_gather = _make_gather()


@jax.jit
def kernel(cat_idx, table):
    flat_idx = cat_idx.reshape(-1).astype(jnp.int32)
    out = _gather(flat_idx, table)
    return out.reshape(BATCH, FIELDS, EMBED_DIM)


# speedup vs baseline: 1.0003x; 1.0002x over previous
import functools

import jax.experimental.layout

import jax
import jax.numpy as jnp
from jax import lax
from jax.experimental import pallas as pl
from jax.experimental.pallas import tpu as pltpu
from jax.experimental.pallas import tpu_sc as plsc

NUM_CATS = 1000000
EMBED_DIM = 32
BATCH = 16384
FIELDS = 26
_B = BATCH * FIELDS
_NC = 2
_NS = 16
_NW = _NC * _NS
_BPW = _B // _NW
_NSTR = 4
_RNG = _BPW // _NSTR
_CHUNK = 416
_NCH = _RNG // _CHUNK


def _make_gather():
    mesh = plsc.VectorSubcoreMesh(core_axis_name="c", subcore_axis_name="s")

    @functools.partial(
        pl.kernel,
        mesh=mesh,
        compiler_params=pltpu.CompilerParams(use_tc_tiling_on_sc=False),
        out_type=jax.ShapeDtypeStruct((_B, EMBED_DIM), jnp.float32),
        scratch_types=(
            [pltpu.VMEM((_BPW,), jnp.int32)]
            + [pltpu.VMEM((_CHUNK, EMBED_DIM), jnp.float32)] * (2 * _NSTR)
            + [pltpu.SemaphoreType.DMA] * (4 * _NSTR)
        ),
    )
    def gather_kernel(idx_hbm, table_hbm, out_hbm, idx_v, *rest):
        bufs = [rest[2 * s:2 * s + 2] for s in range(_NSTR)]
        sems = rest[2 * _NSTR:]
        gsem = [sems[2 * s:2 * s + 2] for s in range(_NSTR)]
        ssem = [sems[2 * _NSTR + 2 * s:2 * _NSTR + 2 * s + 2]
                for s in range(_NSTR)]
        wid = lax.axis_index("s") * _NC + lax.axis_index("c")
        base = wid * _BPW
        pltpu.sync_copy(idx_hbm.at[pl.ds(base, _BPW)], idx_v)

        gather = [[None, None] for _ in range(_NSTR)]
        store = [[None, None] for _ in range(_NSTR)]

        def off(s, c):
            return s * _RNG + c * _CHUNK

        for b in range(2):
            for s in range(_NSTR):
                gather[s][b] = pltpu.async_copy(
                    table_hbm.at[idx_v.at[pl.ds(off(s, b), _CHUNK)]],
                    bufs[s][b], gsem[s][b])
        for c in range(_NCH):
            b = c % 2
            for s in range(_NSTR):
                gather[s][b].wait()
                store[s][b] = pltpu.async_copy(
                    bufs[s][b],
                    out_hbm.at[pl.ds(base + off(s, c), _CHUNK)],
                    ssem[s][b])
            nxt = c + 2
            if nxt < _NCH:
                for s in range(_NSTR):
                    store[s][b].wait()
                    gather[s][b] = pltpu.async_copy(
                        table_hbm.at[idx_v.at[pl.ds(off(s, nxt), _CHUNK)]],
                        bufs[s][b], gsem[s][b])
        for c in range(_NCH - 2, _NCH):
            for s in range(_NSTR):
                store[s][c % 2].wait()

    return gather_kernel


_gather = _make_gather()

def _kernel_impl(cat_idx, table):
    flat_idx = cat_idx.reshape(-1).astype(jnp.int32)
    out = _gather(flat_idx, table)
    return out.reshape(BATCH, FIELDS, EMBED_DIM)


@functools.cache
def _jitted():
    fmt = jax.experimental.layout.Format(
        jax.experimental.layout.Layout(major_to_minor=(0, 1, 2), tiling=()),
        jax.sharding.SingleDeviceSharding(jax.devices()[0]))
    return jax.jit(_kernel_impl, out_shardings=fmt)


def kernel(cat_idx, table):
    return _jitted()(cat_idx, table)
